# Initial kernel scaffold; baseline (speedup 1.0000x reference)
import jax
import jax.numpy as jnp
from jax.experimental import pallas as pl


def kernel(target_fea, index, weight, bias):
    raise NotImplementedError("write your pallas kernel here")


# speedup vs baseline: 1.0652x; 1.0652x over previous
import functools

import jax
import jax.numpy as jnp
from jax.experimental import pallas as pl
from jax.experimental.pallas import tpu as pltpu

_EPS = 1e-6
_R = 128


def _cumsum_rows(v):
    k = 1
    n = v.shape[0]
    while k < n:
        z = jnp.zeros((k, 1), v.dtype)
        v = v + jnp.concatenate([z, v[:-k]], axis=0)
        k *= 2
    return v


def _split3(m):
    h = m.astype(jnp.bfloat16)
    r = m - h.astype(jnp.float32)
    mid = r.astype(jnp.bfloat16)
    lo = (r - mid.astype(jnp.float32)).astype(jnp.bfloat16)
    return h, mid, lo


def _onehot(idx_ref, off, wwin):
    idx = idx_ref[0]
    prev = jnp.concatenate([idx[:1], idx[:-1]], axis=0)
    flags = (idx != prev).astype(jnp.int32)
    rel = _cumsum_rows(flags) + off
    col = jax.lax.broadcasted_iota(jnp.int32, (_R, wwin), 1)
    return (rel == col).astype(jnp.bfloat16)


def _stats_body(base_al_ref, off_ref, idx_ref, x_ref, w_ref, fin_ref,
                acc_ref, *, nblocks, wwin):
    b = pl.program_id(0)

    @pl.when(b == 0)
    def _zero():
        acc_ref[...] = jnp.zeros_like(acc_ref)

    @pl.when(b < nblocks)
    def _accumulate():
        onehot = _onehot(idx_ref, off_ref[b], wwin)
        x = x_ref[...]
        d = x.shape[1]
        xh, xm, xl = _split3(x)
        qh, qm, ql = _split3(x * x)
        m = jnp.concatenate([xh, xm, xl, qh, qm, ql,
                             jnp.ones((x.shape[0], d), jnp.bfloat16)], axis=1)
        dn = (((0,), (0,)), ((), ()))
        s = jax.lax.dot_general(onehot, m, dn,
                                preferred_element_type=jnp.float32)
        moments = jnp.concatenate(
            [s[:, :d] + s[:, d:2 * d] + s[:, 2 * d:3 * d],
             s[:, 3 * d:4 * d] + s[:, 4 * d:5 * d] + s[:, 5 * d:6 * d],
             s[:, 6 * d:]], axis=1)
        start = pl.multiple_of(base_al_ref[b], 8)
        acc_ref[pl.ds(start, wwin), :] += moments

    @pl.when(b == nblocks)
    def _finalize():
        d = w_ref.shape[1]
        ssum = acc_ref[:, :d]
        ssq = acc_ref[:, d:2 * d]
        cnt = acc_ref[:, 2 * d:2 * d + 1]
        safe = jnp.maximum(cnt, 1.0)
        mean = ssum / safe
        ssd = jnp.maximum(ssq - mean * ssum, 0.0) + _EPS
        var = ssd / (cnt - 1.0)
        std = jnp.sqrt(jnp.maximum(var, 1e-7))
        invw = w_ref[...] / (std + _EPS)
        mh, mm, ml = _split3(mean)
        ih = invw.astype(jnp.bfloat16)
        il = (invw - ih.astype(jnp.float32)).astype(jnp.bfloat16)
        fin_ref[...] = jnp.concatenate([mh, mm, ml, ih, il], axis=1)


def _norm_body(base_al_ref, off_ref, idx_ref, x_ref, fin_ref, b_ref, out_ref,
               *, wwin):
    b = pl.program_id(0)
    onehot = _onehot(idx_ref, off_ref[b], wwin)
    start = pl.multiple_of(base_al_ref[b], 8)
    window = fin_ref[pl.ds(start, wwin), :]
    dn = (((1,), (0,)), ((), ()))
    g = jax.lax.dot_general(onehot, window, dn,
                            preferred_element_type=jnp.float32)
    d = x_ref.shape[1]
    mean = g[:, :d] + g[:, d:2 * d] + g[:, 2 * d:3 * d]
    invw = g[:, 3 * d:4 * d] + g[:, 4 * d:]
    out_ref[...] = (x_ref[...] - mean) * invw + b_ref[...]


def _crystal_norm(target_fea, index, weight, bias, num_segments,
                  interpret=False):
    n, d = target_fea.shape
    nblocks = n // _R
    wwin = _R + 8
    s_pad = ((num_segments + wwin + 7) // 8) * 8

    boundary = jnp.concatenate([
        jnp.zeros((1,), jnp.int32),
        (index[1:] != index[:-1]).astype(jnp.int32)])
    rank = jnp.cumsum(boundary, dtype=jnp.int32)
    rank_base = rank[::_R]
    base_al = rank_base - (rank_base % 8)
    off = rank_base - base_al

    idx3 = index.reshape(nblocks, _R, 1)
    w2 = weight.reshape(1, d).astype(jnp.float32)
    b2 = bias.reshape(1, d).astype(jnp.float32)

    stats_spec = pltpu.PrefetchScalarGridSpec(
        num_scalar_prefetch=2,
        grid=(nblocks + 1,),
        in_specs=[
            pl.BlockSpec((1, _R, 1), lambda b, *_: (jnp.minimum(b, nblocks - 1), 0, 0)),
            pl.BlockSpec((_R, d), lambda b, *_: (jnp.minimum(b, nblocks - 1), 0)),
            pl.BlockSpec((1, d), lambda b, *_: (0, 0)),
        ],
        out_specs=pl.BlockSpec((s_pad, 5 * d), lambda b, *_: (0, 0)),
        scratch_shapes=[pltpu.VMEM((s_pad, 2 * d + 128), jnp.float32)],
    )
    fin = pl.pallas_call(
        functools.partial(_stats_body, nblocks=nblocks, wwin=wwin),
        grid_spec=stats_spec,
        out_shape=jax.ShapeDtypeStruct((s_pad, 5 * d), jnp.bfloat16),
        interpret=interpret,
    )(base_al, off, idx3, target_fea, w2)

    norm_spec = pltpu.PrefetchScalarGridSpec(
        num_scalar_prefetch=2,
        grid=(nblocks,),
        in_specs=[
            pl.BlockSpec((1, _R, 1), lambda b, *_: (b, 0, 0)),
            pl.BlockSpec((_R, d), lambda b, *_: (b, 0)),
            pl.BlockSpec((s_pad, 5 * d), lambda b, *_: (0, 0)),
            pl.BlockSpec((1, d), lambda b, *_: (0, 0)),
        ],
        out_specs=pl.BlockSpec((_R, d), lambda b, *_: (b, 0)),
    )
    return pl.pallas_call(
        functools.partial(_norm_body, wwin=wwin),
        grid_spec=norm_spec,
        out_shape=jax.ShapeDtypeStruct((n, d), jnp.float32),
        interpret=interpret,
    )(base_al, off, idx3, target_fea, fin, b2)


def kernel(target_fea, index, weight, bias):
    return _crystal_norm(target_fea, index, weight, bias, 10000)
